# baseline (speedup 1.0000x reference)
import functools

import jax
import jax.numpy as jnp
from jax import lax
from jax.experimental import pallas as pl
from jax.experimental.pallas import tpu as pltpu
from jax.experimental.pallas import tpu_sc as plsc

OUT_ROWS = 576
_NT = 16


@functools.lru_cache(maxsize=None)
def _build_a_fn(rpt: int, in_rows: int):
    a_flat = OUT_ROWS * in_rows
    slice_len = a_flat // _NT
    zch = slice_len // 8
    mesh = plsc.VectorSubcoreMesh(core_axis_name="c", subcore_axis_name="s")

    def body(seg_hbm, idx_hbm, scale_hbm, a_hbm,
             seg_v, idx_v, scale_v, flat_v, zero_v, a_sh, sem):
        cid = lax.axis_index("c")
        tid = lax.axis_index("s")

        @pl.when(cid == 0)
        def _work():
            def zfill(i, carry):
                zero_v[pl.ds(i * 16, 16)] = jnp.zeros((16,), jnp.float32)
                return carry
            lax.fori_loop(0, zch // 16, zfill, 0)
            for j in range(8):
                pltpu.sync_copy(zero_v,
                                a_sh.at[pl.ds(tid * slice_len + j * zch, zch)])

            pltpu.sync_copy(seg_hbm.at[tid], seg_v)
            pltpu.sync_copy(idx_hbm.at[tid], idx_v)
            pltpu.sync_copy(scale_hbm.at[tid], scale_v)

            for r in range(rpt):
                for c in range(8):
                    sl = pl.ds(c * 16, 16)
                    flat_v[r, sl] = idx_v[r, sl] * OUT_ROWS + seg_v[r, sl]

            plsc.subcore_barrier()

            descs = [
                pltpu.async_copy(scale_v.at[r], a_sh.at[flat_v.at[r]], sem,
                                 add=True)
                for r in range(rpt)
            ]
            for d in descs:
                d.wait()

            plsc.subcore_barrier()

            off = tid * slice_len
            pltpu.sync_copy(a_sh.at[pl.ds(off, slice_len)],
                            a_hbm.at[pl.ds(off, slice_len)])

    return pl.kernel(
        body,
        out_type=jax.ShapeDtypeStruct((a_flat,), jnp.float32),
        mesh=mesh,
        scratch_types=[
            pltpu.VMEM((rpt, 128), jnp.int32),
            pltpu.VMEM((rpt, 128), jnp.int32),
            pltpu.VMEM((rpt, 128), jnp.float32),
            pltpu.VMEM((rpt, 128), jnp.int32),
            pltpu.VMEM((zch,), jnp.float32),
            pltpu.VMEM_SHARED((a_flat,), jnp.float32),
            pltpu.SemaphoreType.DMA,
        ],
    )


@functools.lru_cache(maxsize=None)
def _matmul_fn(b: int, in_rows: int, ch: int, g: int):
    def body(at_ref, x_ref, o_ref):
        v = x_ref[...].astype(jnp.bfloat16).reshape(g, in_rows, ch)
        vt = jnp.transpose(v, (0, 2, 1)).reshape(g * ch, in_rows)
        rr = jnp.dot(vt, at_ref[...], preferred_element_type=jnp.float32)
        o_ref[...] = jnp.transpose(
            rr.reshape(g, ch, OUT_ROWS), (0, 2, 1)).reshape(g, OUT_ROWS * ch)

    return pl.pallas_call(
        body,
        grid=(b // g,),
        in_specs=[
            pl.BlockSpec((in_rows, OUT_ROWS), lambda i: (0, 0)),
            pl.BlockSpec((g, in_rows * ch), lambda i: (i, 0)),
        ],
        out_specs=pl.BlockSpec((g, OUT_ROWS * ch), lambda i: (i, 0)),
        out_shape=jax.ShapeDtypeStruct((b, OUT_ROWS * ch), jnp.float32),
    )


def kernel(input, scale, index, seg_out):
    b, in_rows, ch = input.shape
    k = scale.shape[0]
    chunk = _NT * 128
    kp = -(-k // chunk) * chunk
    pad = kp - k
    rpt = kp // chunk
    seg_p = jnp.pad(seg_out, (0, pad)).reshape(_NT, rpt, 128)
    idx_p = jnp.pad(index, (0, pad)).reshape(_NT, rpt, 128)
    scale_p = jnp.pad(scale, (0, pad)).reshape(_NT, rpt, 128)
    at_flat = _build_a_fn(rpt, in_rows)(seg_p, idx_p, scale_p)
    at_mat = at_flat.reshape(in_rows, OUT_ROWS).astype(jnp.bfloat16)
    x2 = input.reshape(b, in_rows * ch)
    out2 = _matmul_fn(b, in_rows, ch, 32)(at_mat, x2)
    return out2.reshape(b, OUT_ROWS, ch)


# speedup vs baseline: 1.2743x; 1.2743x over previous
import functools

import jax
import jax.numpy as jnp
from jax import lax
from jax.experimental import pallas as pl
from jax.experimental.pallas import tpu as pltpu
from jax.experimental.pallas import tpu_sc as plsc

OUT_ROWS = 576
_NT = 16


@functools.lru_cache(maxsize=None)
def _build_a_fn(rpt: int, in_rows: int):
    a_flat = OUT_ROWS * in_rows
    slice_len = a_flat // _NT
    zch = slice_len // 8
    mesh = plsc.VectorSubcoreMesh(core_axis_name="c", subcore_axis_name="s")

    def body(seg_hbm, idx_hbm, scale_hbm, a_hbm,
             seg_v, idx_v, scale_v, flat_v, zero_v, a_sh, sem):
        cid = lax.axis_index("c")
        tid = lax.axis_index("s")

        @pl.when(cid == 0)
        def _work():
            def zfill(i, carry):
                zero_v[pl.ds(i * 16, 16)] = jnp.zeros((16,), jnp.float32)
                return carry
            lax.fori_loop(0, zch // 16, zfill, 0)
            for j in range(8):
                pltpu.sync_copy(zero_v,
                                a_sh.at[pl.ds(tid * slice_len + j * zch, zch)])

            pltpu.sync_copy(seg_hbm.at[tid], seg_v)
            pltpu.sync_copy(idx_hbm.at[tid], idx_v)
            pltpu.sync_copy(scale_hbm.at[tid], scale_v)

            for r in range(rpt):
                for c in range(8):
                    sl = pl.ds(c * 16, 16)
                    flat_v[r, sl] = seg_v[r, sl] * in_rows + idx_v[r, sl]

            plsc.subcore_barrier()

            descs = [
                pltpu.async_copy(scale_v.at[r], a_sh.at[flat_v.at[r]], sem,
                                 add=True)
                for r in range(rpt)
            ]
            for d in descs:
                d.wait()

            plsc.subcore_barrier()

            off = tid * slice_len
            pltpu.sync_copy(a_sh.at[pl.ds(off, slice_len)],
                            a_hbm.at[pl.ds(off, slice_len)])

    return pl.kernel(
        body,
        out_type=jax.ShapeDtypeStruct((a_flat,), jnp.float32),
        mesh=mesh,
        scratch_types=[
            pltpu.VMEM((rpt, 128), jnp.int32),
            pltpu.VMEM((rpt, 128), jnp.int32),
            pltpu.VMEM((rpt, 128), jnp.float32),
            pltpu.VMEM((rpt, 128), jnp.int32),
            pltpu.VMEM((zch,), jnp.float32),
            pltpu.VMEM_SHARED((a_flat,), jnp.float32),
            pltpu.SemaphoreType.DMA,
        ],
    )


@functools.lru_cache(maxsize=None)
def _matmul_fn(in_rows: int, n_total: int, n_blk: int):
    def body(a_ref, x_ref, o_ref):
        o_ref[...] = jnp.dot(a_ref[...], x_ref[...],
                             preferred_element_type=jnp.float32)

    return pl.pallas_call(
        body,
        grid=(n_total // n_blk,),
        in_specs=[
            pl.BlockSpec((OUT_ROWS, in_rows), lambda i: (0, 0)),
            pl.BlockSpec((in_rows, n_blk), lambda i: (0, i)),
        ],
        out_specs=pl.BlockSpec((OUT_ROWS, n_blk), lambda i: (0, i)),
        out_shape=jax.ShapeDtypeStruct((OUT_ROWS, n_total), jnp.float32),
    )


def kernel(input, scale, index, seg_out):
    b, in_rows, ch = input.shape
    k = scale.shape[0]
    chunk = _NT * 128
    kp = -(-k // chunk) * chunk
    pad = kp - k
    rpt = kp // chunk
    seg_p = jnp.pad(seg_out, (0, pad)).reshape(_NT, rpt, 128)
    idx_p = jnp.pad(index, (0, pad)).reshape(_NT, rpt, 128)
    scale_p = jnp.pad(scale, (0, pad)).reshape(_NT, rpt, 128)
    at_flat = _build_a_fn(rpt, in_rows)(seg_p, idx_p, scale_p)
    a_mat = at_flat.reshape(OUT_ROWS, in_rows).astype(jnp.bfloat16)
    n_chunks = 4
    bc = b // n_chunks
    outs = []
    for j in range(n_chunks):
        xj = jax.lax.slice_in_dim(input, j * bc, (j + 1) * bc, axis=0)
        xt = jnp.swapaxes(xj, 0, 1).astype(jnp.bfloat16).reshape(in_rows,
                                                                 bc * ch)
        out_t = _matmul_fn(in_rows, bc * ch, 2048)(a_mat, xt)
        outs.append(jnp.swapaxes(out_t.reshape(OUT_ROWS, bc, ch), 0, 1))
    return jnp.concatenate(outs, axis=0)


# speedup vs baseline: 1.7148x; 1.3457x over previous
import functools

import jax
import jax.numpy as jnp
from jax import lax
from jax.experimental import pallas as pl
from jax.experimental.pallas import tpu as pltpu
from jax.experimental.pallas import tpu_sc as plsc

OUT_ROWS = 576
_NT = 16


@functools.lru_cache(maxsize=None)
def _build_a_fn(rpt: int, in_rows: int):
    a_flat = OUT_ROWS * in_rows
    slice_len = a_flat // _NT
    zch = slice_len // 8
    mesh = plsc.VectorSubcoreMesh(core_axis_name="c", subcore_axis_name="s")

    def body(seg_hbm, idx_hbm, scale_hbm, a_hbm,
             seg_v, idx_v, scale_v, flat_v, zero_v, a_sh, sem):
        cid = lax.axis_index("c")
        tid = lax.axis_index("s")

        @pl.when(cid == 0)
        def _work():
            def zfill(i, carry):
                zero_v[pl.ds(i * 16, 16)] = jnp.zeros((16,), jnp.float32)
                return carry
            lax.fori_loop(0, zch // 16, zfill, 0)
            for j in range(8):
                pltpu.sync_copy(zero_v,
                                a_sh.at[pl.ds(tid * slice_len + j * zch, zch)])

            pltpu.sync_copy(seg_hbm.at[tid], seg_v)
            pltpu.sync_copy(idx_hbm.at[tid], idx_v)
            pltpu.sync_copy(scale_hbm.at[tid], scale_v)

            for r in range(rpt):
                for c in range(8):
                    sl = pl.ds(c * 16, 16)
                    flat_v[r, sl] = seg_v[r, sl] * in_rows + idx_v[r, sl]

            plsc.subcore_barrier()

            descs = [
                pltpu.async_copy(scale_v.at[r], a_sh.at[flat_v.at[r]], sem,
                                 add=True)
                for r in range(rpt)
            ]
            for d in descs:
                d.wait()

            plsc.subcore_barrier()

            off = tid * slice_len
            pltpu.sync_copy(a_sh.at[pl.ds(off, slice_len)],
                            a_hbm.at[pl.ds(off, slice_len)])

    return pl.kernel(
        body,
        out_type=jax.ShapeDtypeStruct((a_flat,), jnp.float32),
        mesh=mesh,
        scratch_types=[
            pltpu.VMEM((rpt, 128), jnp.int32),
            pltpu.VMEM((rpt, 128), jnp.int32),
            pltpu.VMEM((rpt, 128), jnp.float32),
            pltpu.VMEM((rpt, 128), jnp.int32),
            pltpu.VMEM((zch,), jnp.float32),
            pltpu.VMEM_SHARED((a_flat,), jnp.float32),
            pltpu.SemaphoreType.DMA,
        ],
    )


@functools.lru_cache(maxsize=None)
def _matmul_fn(in_rows: int, n_total: int, n_blk: int):
    def body(a_ref, x_ref, o_ref):
        o_ref[...] = jnp.dot(a_ref[...], x_ref[...],
                             preferred_element_type=jnp.float32)

    return pl.pallas_call(
        body,
        grid=(n_total // n_blk,),
        in_specs=[
            pl.BlockSpec((OUT_ROWS, in_rows), lambda i: (0, 0)),
            pl.BlockSpec((in_rows, n_blk), lambda i: (0, i)),
        ],
        out_specs=pl.BlockSpec((OUT_ROWS, n_blk), lambda i: (0, i)),
        out_shape=jax.ShapeDtypeStruct((OUT_ROWS, n_total), jnp.float32),
    )


def kernel(input, scale, index, seg_out):
    b, in_rows, ch = input.shape
    k = scale.shape[0]
    chunk = _NT * 128
    kp = -(-k // chunk) * chunk
    pad = kp - k
    rpt = kp // chunk
    seg_p = jnp.pad(seg_out, (0, pad)).reshape(_NT, rpt, 128)
    idx_p = jnp.pad(index, (0, pad)).reshape(_NT, rpt, 128)
    scale_p = jnp.pad(scale, (0, pad)).reshape(_NT, rpt, 128)
    at_flat = _build_a_fn(rpt, in_rows)(seg_p, idx_p, scale_p)
    a_mat = at_flat.reshape(OUT_ROWS, in_rows)
    xt = jnp.swapaxes(input, 0, 1).reshape(in_rows, b * ch)
    out_t = _matmul_fn(in_rows, b * ch, 1024)(a_mat, xt)
    return jnp.swapaxes(out_t.reshape(OUT_ROWS, b, ch), 0, 1)


# speedup vs baseline: 1.8877x; 1.1008x over previous
import functools

import jax
import jax.numpy as jnp
from jax import lax
from jax.experimental import pallas as pl
from jax.experimental.pallas import tpu as pltpu
from jax.experimental.pallas import tpu_sc as plsc

OUT_ROWS = 576
_NT = 16


@functools.lru_cache(maxsize=None)
def _build_a_fn(rpt: int, in_rows: int):
    a_flat = OUT_ROWS * in_rows
    slice_len = a_flat // _NT
    zch = slice_len // 8
    mesh = plsc.VectorSubcoreMesh(core_axis_name="c", subcore_axis_name="s")

    def body(seg_hbm, idx_hbm, scale_hbm, a_hbm,
             seg_v, idx_v, scale_v, flat_v, zero_v, a_sh, sem):
        cid = lax.axis_index("c")
        tid = lax.axis_index("s")

        @pl.when(cid == 0)
        def _work():
            def zfill(i, carry):
                zero_v[pl.ds(i * 16, 16)] = jnp.zeros((16,), jnp.float32)
                return carry
            lax.fori_loop(0, zch // 16, zfill, 0)
            for j in range(8):
                pltpu.sync_copy(zero_v,
                                a_sh.at[pl.ds(tid * slice_len + j * zch, zch)])

            pltpu.sync_copy(seg_hbm.at[tid], seg_v)
            pltpu.sync_copy(idx_hbm.at[tid], idx_v)
            pltpu.sync_copy(scale_hbm.at[tid], scale_v)

            for r in range(rpt):
                for c in range(8):
                    sl = pl.ds(c * 16, 16)
                    flat_v[r, sl] = seg_v[r, sl] * in_rows + idx_v[r, sl]

            plsc.subcore_barrier()

            descs = [
                pltpu.async_copy(scale_v.at[r], a_sh.at[flat_v.at[r]], sem,
                                 add=True)
                for r in range(rpt)
            ]
            for d in descs:
                d.wait()

            plsc.subcore_barrier()

            off = tid * slice_len
            pltpu.sync_copy(a_sh.at[pl.ds(off, slice_len)],
                            a_hbm.at[pl.ds(off, slice_len)])

    return pl.kernel(
        body,
        out_type=jax.ShapeDtypeStruct((a_flat,), jnp.float32),
        mesh=mesh,
        scratch_types=[
            pltpu.VMEM((rpt, 128), jnp.int32),
            pltpu.VMEM((rpt, 128), jnp.int32),
            pltpu.VMEM((rpt, 128), jnp.float32),
            pltpu.VMEM((rpt, 128), jnp.int32),
            pltpu.VMEM((zch,), jnp.float32),
            pltpu.VMEM_SHARED((a_flat,), jnp.float32),
            pltpu.SemaphoreType.DMA,
        ],
    )


@functools.lru_cache(maxsize=None)
def _matmul_fn(in_rows: int, n_total: int, n_blk: int):
    def body(a_ref, x_ref, o_ref):
        o_ref[...] = jnp.dot(a_ref[...], x_ref[...],
                             preferred_element_type=jnp.float32)

    return pl.pallas_call(
        body,
        grid=(n_total // n_blk,),
        in_specs=[
            pl.BlockSpec((OUT_ROWS, in_rows), lambda i: (0, 0)),
            pl.BlockSpec((in_rows, n_blk), lambda i: (0, i)),
        ],
        out_specs=pl.BlockSpec((OUT_ROWS, n_blk), lambda i: (0, i)),
        out_shape=jax.ShapeDtypeStruct((OUT_ROWS, n_total), jnp.float32),
    )


def kernel(input, scale, index, seg_out):
    b, in_rows, ch = input.shape
    k = scale.shape[0]
    chunk = _NT * 128
    kp = -(-k // chunk) * chunk
    pad = kp - k
    rpt = kp // chunk
    seg_p = jnp.pad(seg_out, (0, pad)).reshape(_NT, rpt, 128)
    idx_p = jnp.pad(index, (0, pad)).reshape(_NT, rpt, 128)
    scale_p = jnp.pad(scale, (0, pad)).reshape(_NT, rpt, 128)
    at_flat = _build_a_fn(rpt, in_rows)(seg_p, idx_p, scale_p)
    a_mat = at_flat.reshape(OUT_ROWS, in_rows).astype(jnp.bfloat16)
    xt = jnp.swapaxes(input, 0, 1).astype(jnp.bfloat16).reshape(in_rows,
                                                                b * ch)
    out_t = _matmul_fn(in_rows, b * ch, 2048)(a_mat, xt)
    return jnp.swapaxes(out_t.reshape(OUT_ROWS, b, ch), 0, 1)
